# baseline (speedup 1.0000x reference)
import functools

import jax
import jax.numpy as jnp
from jax import lax
from jax.experimental import pallas as pl
from jax.experimental.pallas import tpu as pltpu
from jax.experimental.pallas import tpu_sc as plsc

N = 10000
E = 320000
D = 128
H = 128
H8 = 16
G = 64
OUTC = 2

NCORE = 2
NSUB = 16
NW = NCORE * NSUB
CHUNK = 128
NCHUNKS = 80
WIN = 8
NWIN = NCHUNKS // WIN
E_PAD = NW * NCHUNKS * CHUNK
N_PAD = 10240
ROWS_PER_SUB = N_PAD // NSUB

def _leaky(v):
    return jnp.where(v >= 0, v, 0.2 * v)


def _bn_eval(v, g, b):
    return v / jnp.sqrt(1.0 + 1e-5) * g + b


def _deg_body(dst3, ones_c, zeros_n, out, dst_v, ones_v, deg_sh):
    cid = lax.axis_index("c")
    sid = lax.axis_index("s")
    wid = sid * NCORE + cid
    pltpu.sync_copy(dst3.at[wid], dst_v)
    pltpu.sync_copy(ones_c, ones_v)

    @pl.when(sid == 0)
    def _():
        pltpu.sync_copy(zeros_n, deg_sh)

    plsc.subcore_barrier()

    def body(j, carry):
        pltpu.sync_copy(ones_v, deg_sh.at[dst_v.at[j]], add=True)
        return carry

    lax.fori_loop(0, NCHUNKS, body, 0)
    plsc.subcore_barrier()

    @pl.when(sid == 0)
    def _():
        pltpu.sync_copy(deg_sh, out.at[cid])


@functools.cache
def _deg_call():
    return pl.kernel(
        _deg_body,
        out_type=jax.ShapeDtypeStruct((NCORE, N_PAD), jnp.float32),
        mesh=plsc.VectorSubcoreMesh(core_axis_name="c", subcore_axis_name="s"),
        scratch_types=[
            pltpu.VMEM((NCHUNKS, CHUNK), jnp.int32),
            pltpu.VMEM((CHUNK,), jnp.float32),
            pltpu.VMEM_SHARED((N_PAD,), jnp.float32),
        ],
    )


def _agg_body(hp, src2, dst3, zeros_blk, out, src_v, dwin0, dwin1, rows0, rows1,
              dsem0, dsem1, gsem0, gsem1, ssem0, ssem1, agg_sh):
    cid = lax.axis_index("c")
    sid = lax.axis_index("s")
    wid = sid * NCORE + cid
    pltpu.sync_copy(src2.at[wid], src_v)
    dwin = (dwin0, dwin1)
    dsem = (dsem0, dsem1)
    rows = (rows0, rows1)
    gsem = (gsem0, gsem1)
    ssem = (ssem0, ssem1)
    pltpu.async_copy(dst3.at[wid].at[pl.ds(0, WIN)], dwin[0], dsem[0])
    pltpu.sync_copy(zeros_blk, agg_sh.at[pl.ds(sid * ROWS_PER_SUB, ROWS_PER_SUB)])
    plsc.subcore_barrier()
    pltpu.async_copy(hp.at[src_v.at[pl.ds(0, CHUNK)]], rows[0], gsem[0])

    def outer(t, carry):
        for ww in range(2):
            w = 2 * t + ww

            @pl.when(w >= 1)
            def _():
                pltpu.make_async_copy(rows[1], agg_sh.at[dwin[1 - ww].at[0]],
                                      ssem[1]).wait()

            @pl.when(w + 1 < NWIN)
            def _():
                pltpu.async_copy(dst3.at[wid].at[pl.ds((w + 1) * WIN, WIN)],
                                 dwin[1 - ww], dsem[1 - ww])

            pltpu.make_async_copy(dst3.at[wid].at[pl.ds(0, WIN)],
                                  dwin[ww], dsem[ww]).wait()
            for b in range(WIN):
                j = w * WIN + b

                @pl.when((j >= 1) & (j + 1 < NCHUNKS) & (b != 0))
                def _():
                    pltpu.make_async_copy(rows[1 - b % 2],
                                          agg_sh.at[dwin[ww].at[b]],
                                          ssem[1 - b % 2]).wait()

                @pl.when(j + 1 < NCHUNKS)
                def _():
                    pltpu.async_copy(
                        hp.at[src_v.at[pl.ds((j + 1) * CHUNK, CHUNK)]],
                        rows[1 - b % 2], gsem[1 - b % 2])

                pltpu.make_async_copy(hp.at[src_v.at[pl.ds(j * CHUNK, CHUNK)]],
                                      rows[b % 2], gsem[b % 2]).wait()
                pltpu.async_copy(rows[b % 2], agg_sh.at[dwin[ww].at[b]],
                                 ssem[b % 2], add=True)
        return carry

    lax.fori_loop(0, NWIN // 2, outer, 0)
    pltpu.make_async_copy(rows[0], agg_sh.at[dwin[1].at[0]], ssem[0]).wait()
    pltpu.make_async_copy(rows[1], agg_sh.at[dwin[1].at[1]], ssem[1]).wait()
    plsc.subcore_barrier()
    pltpu.sync_copy(
        agg_sh.at[pl.ds(sid * ROWS_PER_SUB, ROWS_PER_SUB)],
        out.at[cid].at[pl.ds(sid * ROWS_PER_SUB, ROWS_PER_SUB)],
    )


@functools.cache
def _agg_call():
    return pl.kernel(
        _agg_body,
        out_type=jax.ShapeDtypeStruct((NCORE, N_PAD, D), jnp.float32),
        mesh=plsc.VectorSubcoreMesh(core_axis_name="c", subcore_axis_name="s"),
        scratch_types=[
            pltpu.VMEM((NCHUNKS * CHUNK,), jnp.int32),
            pltpu.VMEM((WIN, CHUNK), jnp.int32),
            pltpu.VMEM((WIN, CHUNK), jnp.int32),
            pltpu.VMEM((CHUNK, D), jnp.float32),
            pltpu.VMEM((CHUNK, D), jnp.float32),
            pltpu.SemaphoreType.DMA,
            pltpu.SemaphoreType.DMA,
            pltpu.SemaphoreType.DMA,
            pltpu.SemaphoreType.DMA,
            pltpu.SemaphoreType.DMA,
            pltpu.SemaphoreType.DMA,
            pltpu.VMEM_SHARED((N_PAD, D), jnp.float32),
        ],
    )


def _tc1_body(x_ref, w1_ref, l0w_ref, l0b_ref, degp_ref, hp1_ref, h1_ref, ft_ref):
    deg = degp_ref[0] + degp_ref[1] + 1.0
    norm = lax.rsqrt(deg)
    x = x_ref[...]
    h1 = jnp.dot(x, w1_ref[...], preferred_element_type=jnp.float32)
    h1_ref[...] = h1
    hp1_ref[:N, :] = h1 * norm
    hp1_ref[N:, :] = jnp.zeros((N_PAD - N, H), jnp.float32)
    ft_ref[...] = jnp.maximum(
        jnp.dot(x, l0w_ref[...], preferred_element_type=jnp.float32) + l0b_ref[...],
        0.0,
    )


_tc1_call = pl.pallas_call(
    _tc1_body,
    out_shape=(
        jax.ShapeDtypeStruct((N_PAD, H), jnp.float32),
        jax.ShapeDtypeStruct((N, H), jnp.float32),
        jax.ShapeDtypeStruct((N, H8), jnp.float32),
    ),
)


def _tc2_body(aggp_ref, h1_ref, degp_ref, b1_ref, w2_ref, hp2_ref, h2_ref):
    deg = degp_ref[0] + degp_ref[1] + 1.0
    norm = lax.rsqrt(deg)
    inv = 1.0 / deg
    h1 = h1_ref[...]
    agg = (aggp_ref[0, :N, :] + aggp_ref[1, :N, :]) * norm + h1 * inv + b1_ref[...]
    h1f = _leaky(agg)
    h2 = jnp.dot(h1f, w2_ref[...], preferred_element_type=jnp.float32)
    h2_ref[...] = h2
    hp2_ref[:N, :] = h2 * norm
    hp2_ref[N:, :] = jnp.zeros((N_PAD - N, H), jnp.float32)


_tc2_call = pl.pallas_call(
    _tc2_body,
    out_shape=(
        jax.ShapeDtypeStruct((N_PAD, H), jnp.float32),
        jax.ShapeDtypeStruct((N, H), jnp.float32),
    ),
)


def _tc3_body(aggp_ref, h2_ref, degp_ref, b2_ref, batch_ref, ft_ref,
              l1w_ref, l1b_ref, bn1g_ref, bn1b_ref,
              l2w_ref, l2b_ref, bn2g_ref, bn2b_ref, emb_ref, out_ref):
    deg = degp_ref[0] + degp_ref[1] + 1.0
    norm = lax.rsqrt(deg)
    inv = 1.0 / deg
    h2 = h2_ref[...]
    h2f = _leaky((aggp_ref[0, :N, :] + aggp_ref[1, :N, :]) * norm + h2 * inv + b2_ref[...])

    seg = lax.broadcasted_iota(jnp.int32, (G, N), 0)
    pmat = (seg == batch_ref[...]).astype(jnp.float32)
    counts = jnp.sum(pmat, axis=1, keepdims=True)
    denom = jnp.maximum(counts, 1.0)
    pooled = jnp.dot(pmat, h2f, preferred_element_type=jnp.float32) / denom
    ft_pool = jnp.dot(pmat, ft_ref[...], preferred_element_type=jnp.float32) / denom

    z = _leaky(_bn_eval(
        jnp.dot(pooled, l1w_ref[...], preferred_element_type=jnp.float32)
        + l1b_ref[...],
        bn1g_ref[...], bn1b_ref[...]))

    big = counts >= 10.0
    x1 = jnp.where(big, emb_ref[1:2, :], emb_ref[0:1, :])
    x1 = _leaky(_bn_eval(x1, bn2g_ref[...], bn2b_ref[...]))
    x1 = (1.0 / (1.0 + jnp.exp(-x1))) * ft_pool

    cat = jnp.concatenate([z, x1], axis=1)
    out_ref[...] = (
        jnp.dot(cat, l2w_ref[...], preferred_element_type=jnp.float32)
        + l2b_ref[...]
    )


_tc3_call = pl.pallas_call(
    _tc3_body,
    out_shape=jax.ShapeDtypeStruct((G, OUTC), jnp.float32),
)


def kernel(x, edge_index, batch, W_conv1, b_conv1, W_conv2, b_conv2,
           lin0_W, lin0_b, lin1_W, lin1_b, lin2_W, lin2_b,
           bn1_g, bn1_b, bn2_g, bn2_b, emb):
    pad = N + (jnp.arange(E_PAD - E, dtype=jnp.int32) % (N_PAD - N))
    src2 = jnp.concatenate([edge_index[0], pad]).reshape(NW, NCHUNKS * CHUNK)
    dst3 = jnp.concatenate([edge_index[1], pad]).reshape(NW, NCHUNKS, CHUNK)
    ones_c = jnp.ones((CHUNK,), jnp.float32)
    zeros_n = jnp.zeros((N_PAD,), jnp.float32)
    zeros_blk = jnp.zeros((ROWS_PER_SUB, D), jnp.float32)

    degp = _deg_call()(dst3, ones_c, zeros_n)[:, :N].reshape(NCORE, N, 1)

    hp1, h1, ft = _tc1_call(x, W_conv1, lin0_W, lin0_b.reshape(1, H8), degp)
    agg1 = _agg_call()(hp1, src2, dst3, zeros_blk)
    hp2, h2 = _tc2_call(agg1, h1, degp, b_conv1.reshape(1, H), W_conv2)
    agg2 = _agg_call()(hp2, src2, dst3, zeros_blk)
    return _tc3_call(agg2, h2, degp, b_conv2.reshape(1, H),
                     batch.reshape(1, N), ft,
                     lin1_W, lin1_b.reshape(1, H8),
                     bn1_g.reshape(1, H8), bn1_b.reshape(1, H8),
                     lin2_W, lin2_b.reshape(1, OUTC),
                     bn2_g.reshape(1, H8), bn2_b.reshape(1, H8), emb)


# speedup vs baseline: 1.0091x; 1.0091x over previous
import functools

import jax
import jax.numpy as jnp
from jax import lax
from jax.experimental import pallas as pl
from jax.experimental.pallas import tpu as pltpu
from jax.experimental.pallas import tpu_sc as plsc

N = 10000
E = 320000
D = 128
H = 128
H8 = 16
G = 64
OUTC = 2

NCORE = 2
NSUB = 16
NW = NCORE * NSUB
CHUNK = 128
NCHUNKS = 80
WIN = 8
NWIN = NCHUNKS // WIN
E_PAD = NW * NCHUNKS * CHUNK
N_PAD = 10240
ROWS_PER_SUB = N_PAD // NSUB

def _leaky(v):
    return jnp.where(v >= 0, v, 0.2 * v)


def _bn_eval(v, g, b):
    return v / jnp.sqrt(1.0 + 1e-5) * g + b


def _deg_body(dst3, ones_c, zeros_n, out, dst_v, ones_v, deg_sh):
    cid = lax.axis_index("c")
    sid = lax.axis_index("s")
    wid = sid * NCORE + cid
    pltpu.sync_copy(dst3.at[wid], dst_v)
    pltpu.sync_copy(ones_c, ones_v)

    @pl.when(sid == 0)
    def _():
        pltpu.sync_copy(zeros_n, deg_sh)

    plsc.subcore_barrier()

    def body(j, carry):
        pltpu.sync_copy(ones_v, deg_sh.at[dst_v.at[j]], add=True)
        return carry

    lax.fori_loop(0, NCHUNKS, body, 0)
    plsc.subcore_barrier()

    @pl.when(sid == 0)
    def _():
        pltpu.sync_copy(deg_sh, out.at[cid])


@functools.cache
def _deg_call():
    return pl.kernel(
        _deg_body,
        out_type=jax.ShapeDtypeStruct((NCORE, N_PAD), jnp.float32),
        mesh=plsc.VectorSubcoreMesh(core_axis_name="c", subcore_axis_name="s"),
        scratch_types=[
            pltpu.VMEM((NCHUNKS, CHUNK), jnp.int32),
            pltpu.VMEM((CHUNK,), jnp.float32),
            pltpu.VMEM_SHARED((N_PAD,), jnp.float32),
        ],
    )


def _agg_body(hp, src2, dst3, zeros_blk, out, src_v, dwin0, dwin1, rows0, rows1,
              dsem0, dsem1, gsem0, gsem1, agg_sh):
    cid = lax.axis_index("c")
    sid = lax.axis_index("s")
    wid = sid * NCORE + cid
    pltpu.sync_copy(src2.at[wid], src_v)
    dwin = (dwin0, dwin1)
    dsem = (dsem0, dsem1)
    rows = (rows0, rows1)
    gsem = (gsem0, gsem1)
    pltpu.async_copy(dst3.at[wid].at[pl.ds(0, WIN)], dwin[0], dsem[0])
    pltpu.sync_copy(zeros_blk, agg_sh.at[pl.ds(sid * ROWS_PER_SUB, ROWS_PER_SUB)])
    plsc.subcore_barrier()
    pltpu.async_copy(hp.at[src_v.at[pl.ds(0, CHUNK)]], rows[0], gsem[0])

    def outer(t, carry):
        for ww in range(2):
            w = 2 * t + ww

            @pl.when(w + 1 < NWIN)
            def _():
                pltpu.async_copy(dst3.at[wid].at[pl.ds((w + 1) * WIN, WIN)],
                                 dwin[1 - ww], dsem[1 - ww])

            pltpu.make_async_copy(dst3.at[wid].at[pl.ds(0, WIN)],
                                  dwin[ww], dsem[ww]).wait()
            for b in range(WIN):
                j = w * WIN + b

                @pl.when(j + 1 < NCHUNKS)
                def _():
                    pltpu.async_copy(
                        hp.at[src_v.at[pl.ds((j + 1) * CHUNK, CHUNK)]],
                        rows[1 - b % 2], gsem[1 - b % 2])

                pltpu.make_async_copy(hp.at[src_v.at[pl.ds(j * CHUNK, CHUNK)]],
                                      rows[b % 2], gsem[b % 2]).wait()
                pltpu.sync_copy(rows[b % 2], agg_sh.at[dwin[ww].at[b]], add=True)
        return carry

    lax.fori_loop(0, NWIN // 2, outer, 0)
    plsc.subcore_barrier()
    pltpu.sync_copy(
        agg_sh.at[pl.ds(sid * ROWS_PER_SUB, ROWS_PER_SUB)],
        out.at[cid].at[pl.ds(sid * ROWS_PER_SUB, ROWS_PER_SUB)],
    )


@functools.cache
def _agg_call():
    return pl.kernel(
        _agg_body,
        out_type=jax.ShapeDtypeStruct((NCORE, N_PAD, D), jnp.float32),
        mesh=plsc.VectorSubcoreMesh(core_axis_name="c", subcore_axis_name="s"),
        scratch_types=[
            pltpu.VMEM((NCHUNKS * CHUNK,), jnp.int32),
            pltpu.VMEM((WIN, CHUNK), jnp.int32),
            pltpu.VMEM((WIN, CHUNK), jnp.int32),
            pltpu.VMEM((CHUNK, D), jnp.float32),
            pltpu.VMEM((CHUNK, D), jnp.float32),
            pltpu.SemaphoreType.DMA,
            pltpu.SemaphoreType.DMA,
            pltpu.SemaphoreType.DMA,
            pltpu.SemaphoreType.DMA,
            pltpu.VMEM_SHARED((N_PAD, D), jnp.float32),
        ],
    )


def _tc1a_body(x_ref, w1_ref, l0w_ref, l0b_ref, h1_ref, ft_ref):
    x = x_ref[...]
    h1_ref[...] = jnp.dot(x, w1_ref[...], preferred_element_type=jnp.float32)
    ft_ref[...] = jnp.maximum(
        jnp.dot(x, l0w_ref[...], preferred_element_type=jnp.float32) + l0b_ref[...],
        0.0,
    )


_tc1a_call = pl.pallas_call(
    _tc1a_body,
    out_shape=(
        jax.ShapeDtypeStruct((N, H), jnp.float32),
        jax.ShapeDtypeStruct((N, H8), jnp.float32),
    ),
)


def _tc1b_body(h1_ref, degp_ref, hp1_ref):
    deg = degp_ref[0] + degp_ref[1] + 1.0
    norm = lax.rsqrt(deg)
    hp1_ref[:N, :] = h1_ref[...] * norm
    hp1_ref[N:, :] = jnp.zeros((N_PAD - N, H), jnp.float32)


_tc1b_call = pl.pallas_call(
    _tc1b_body,
    out_shape=jax.ShapeDtypeStruct((N_PAD, H), jnp.float32),
)


def _tc2_body(aggp_ref, h1_ref, degp_ref, b1_ref, w2_ref, hp2_ref, h2_ref):
    deg = degp_ref[0] + degp_ref[1] + 1.0
    norm = lax.rsqrt(deg)
    inv = 1.0 / deg
    h1 = h1_ref[...]
    agg = (aggp_ref[0, :N, :] + aggp_ref[1, :N, :]) * norm + h1 * inv + b1_ref[...]
    h1f = _leaky(agg)
    h2 = jnp.dot(h1f, w2_ref[...], preferred_element_type=jnp.float32)
    h2_ref[...] = h2
    hp2_ref[:N, :] = h2 * norm
    hp2_ref[N:, :] = jnp.zeros((N_PAD - N, H), jnp.float32)


_tc2_call = pl.pallas_call(
    _tc2_body,
    out_shape=(
        jax.ShapeDtypeStruct((N_PAD, H), jnp.float32),
        jax.ShapeDtypeStruct((N, H), jnp.float32),
    ),
)


def _tc3_body(aggp_ref, h2_ref, degp_ref, b2_ref, batch_ref, ft_ref,
              l1w_ref, l1b_ref, bn1g_ref, bn1b_ref,
              l2w_ref, l2b_ref, bn2g_ref, bn2b_ref, emb_ref, out_ref):
    deg = degp_ref[0] + degp_ref[1] + 1.0
    norm = lax.rsqrt(deg)
    inv = 1.0 / deg
    h2 = h2_ref[...]
    h2f = _leaky((aggp_ref[0, :N, :] + aggp_ref[1, :N, :]) * norm + h2 * inv + b2_ref[...])

    seg = lax.broadcasted_iota(jnp.int32, (G, N), 0)
    pmat = (seg == batch_ref[...]).astype(jnp.float32)
    counts = jnp.sum(pmat, axis=1, keepdims=True)
    denom = jnp.maximum(counts, 1.0)
    pooled = jnp.dot(pmat, h2f, preferred_element_type=jnp.float32) / denom
    ft_pool = jnp.dot(pmat, ft_ref[...], preferred_element_type=jnp.float32) / denom

    z = _leaky(_bn_eval(
        jnp.dot(pooled, l1w_ref[...], preferred_element_type=jnp.float32)
        + l1b_ref[...],
        bn1g_ref[...], bn1b_ref[...]))

    big = counts >= 10.0
    x1 = jnp.where(big, emb_ref[1:2, :], emb_ref[0:1, :])
    x1 = _leaky(_bn_eval(x1, bn2g_ref[...], bn2b_ref[...]))
    x1 = (1.0 / (1.0 + jnp.exp(-x1))) * ft_pool

    cat = jnp.concatenate([z, x1], axis=1)
    out_ref[...] = (
        jnp.dot(cat, l2w_ref[...], preferred_element_type=jnp.float32)
        + l2b_ref[...]
    )


_tc3_call = pl.pallas_call(
    _tc3_body,
    out_shape=jax.ShapeDtypeStruct((G, OUTC), jnp.float32),
)


def kernel(x, edge_index, batch, W_conv1, b_conv1, W_conv2, b_conv2,
           lin0_W, lin0_b, lin1_W, lin1_b, lin2_W, lin2_b,
           bn1_g, bn1_b, bn2_g, bn2_b, emb):
    pad = N + (jnp.arange(E_PAD - E, dtype=jnp.int32) % (N_PAD - N))
    src2 = jnp.concatenate([edge_index[0], pad]).reshape(NW, NCHUNKS * CHUNK)
    dst3 = jnp.concatenate([edge_index[1], pad]).reshape(NW, NCHUNKS, CHUNK)
    ones_c = jnp.ones((CHUNK,), jnp.float32)
    zeros_n = jnp.zeros((N_PAD,), jnp.float32)
    zeros_blk = jnp.zeros((ROWS_PER_SUB, D), jnp.float32)

    degp = _deg_call()(dst3, ones_c, zeros_n)[:, :N].reshape(NCORE, N, 1)

    h1, ft = _tc1a_call(x, W_conv1, lin0_W, lin0_b.reshape(1, H8))
    hp1 = _tc1b_call(h1, degp)
    agg1 = _agg_call()(hp1, src2, dst3, zeros_blk)
    hp2, h2 = _tc2_call(agg1, h1, degp, b_conv1.reshape(1, H), W_conv2)
    agg2 = _agg_call()(hp2, src2, dst3, zeros_blk)
    return _tc3_call(agg2, h2, degp, b_conv2.reshape(1, H),
                     batch.reshape(1, N), ft,
                     lin1_W, lin1_b.reshape(1, H8),
                     bn1_g.reshape(1, H8), bn1_b.reshape(1, H8),
                     lin2_W, lin2_b.reshape(1, OUTC),
                     bn2_g.reshape(1, H8), bn2_b.reshape(1, H8), emb)


# speedup vs baseline: 1.0302x; 1.0208x over previous
import functools

import jax
import jax.numpy as jnp
from jax import lax
from jax.experimental import pallas as pl
from jax.experimental.pallas import tpu as pltpu
from jax.experimental.pallas import tpu_sc as plsc

N = 10000
E = 320000
D = 128
H = 128
H8 = 16
G = 64
OUTC = 2

NCORE = 2
NSUB = 16
NW = NCORE * NSUB
CHUNK = 128
NCHUNKS = 80
WIN = 8
NWIN = NCHUNKS // WIN
E_PAD = NW * NCHUNKS * CHUNK
N_PAD = 10240
ROWS_PER_SUB = N_PAD // NSUB

def _leaky(v):
    return jnp.where(v >= 0, v, 0.2 * v)


def _bn_eval(v, g, b):
    return v / jnp.sqrt(1.0 + 1e-5) * g + b


def _deg_body(dst3, ones_c, zeros_n, out, dst_v, ones_v, deg_sh):
    cid = lax.axis_index("c")
    sid = lax.axis_index("s")
    wid = sid * NCORE + cid
    pltpu.sync_copy(dst3.at[wid], dst_v)
    pltpu.sync_copy(ones_c, ones_v)

    @pl.when(sid == 0)
    def _():
        pltpu.sync_copy(zeros_n, deg_sh)

    plsc.subcore_barrier()

    def body(j, carry):
        pltpu.sync_copy(ones_v, deg_sh.at[dst_v.at[j]], add=True)
        return carry

    lax.fori_loop(0, NCHUNKS, body, 0)
    plsc.subcore_barrier()

    @pl.when(sid == 0)
    def _():
        pltpu.sync_copy(deg_sh, out.at[cid])


@functools.cache
def _deg_call():
    return pl.kernel(
        _deg_body,
        out_type=jax.ShapeDtypeStruct((NCORE, N_PAD), jnp.float32),
        mesh=plsc.VectorSubcoreMesh(core_axis_name="c", subcore_axis_name="s"),
        scratch_types=[
            pltpu.VMEM((NCHUNKS, CHUNK), jnp.int32),
            pltpu.VMEM((CHUNK,), jnp.float32),
            pltpu.VMEM_SHARED((N_PAD,), jnp.float32),
        ],
    )


def _agg_body(hp, src2, dst3, zeros_blk, out, src_v, dwin0, dwin1, rows0, rows1,
              dsem0, dsem1, gsem0, gsem1, agg_sh):
    cid = lax.axis_index("c")
    sid = lax.axis_index("s")
    wid = sid * NCORE + cid
    pltpu.sync_copy(src2.at[wid], src_v)
    dwin = (dwin0, dwin1)
    dsem = (dsem0, dsem1)
    rows = (rows0, rows1)
    gsem = (gsem0, gsem1)
    pltpu.async_copy(dst3.at[wid].at[pl.ds(0, WIN)], dwin[0], dsem[0])
    pltpu.sync_copy(zeros_blk, agg_sh.at[pl.ds(sid * ROWS_PER_SUB, ROWS_PER_SUB)])
    plsc.subcore_barrier()
    pltpu.async_copy(hp.at[src_v.at[pl.ds(0, CHUNK)]], rows[0], gsem[0])

    def outer(t, carry):
        for ww in range(2):
            w = 2 * t + ww

            @pl.when(w + 1 < NWIN)
            def _():
                pltpu.async_copy(dst3.at[wid].at[pl.ds((w + 1) * WIN, WIN)],
                                 dwin[1 - ww], dsem[1 - ww])

            pltpu.make_async_copy(dst3.at[wid].at[pl.ds(0, WIN)],
                                  dwin[ww], dsem[ww]).wait()
            for b in range(WIN):
                j = w * WIN + b

                @pl.when(j + 1 < NCHUNKS)
                def _():
                    pltpu.async_copy(
                        hp.at[src_v.at[pl.ds((j + 1) * CHUNK, CHUNK)]],
                        rows[1 - b % 2], gsem[1 - b % 2])

                pltpu.make_async_copy(hp.at[src_v.at[pl.ds(j * CHUNK, CHUNK)]],
                                      rows[b % 2], gsem[b % 2]).wait()
                pltpu.sync_copy(rows[b % 2], agg_sh.at[dwin[ww].at[b]], add=True)
        return carry

    lax.fori_loop(0, NWIN // 2, outer, 0)
    plsc.subcore_barrier()
    pltpu.sync_copy(
        agg_sh.at[pl.ds(sid * ROWS_PER_SUB, ROWS_PER_SUB)],
        out.at[cid].at[pl.ds(sid * ROWS_PER_SUB, ROWS_PER_SUB)],
    )


@functools.cache
def _agg_call():
    return pl.kernel(
        _agg_body,
        out_type=jax.ShapeDtypeStruct((NCORE, N_PAD, D), jnp.float32),
        mesh=plsc.VectorSubcoreMesh(core_axis_name="c", subcore_axis_name="s"),
        scratch_types=[
            pltpu.VMEM((NCHUNKS * CHUNK,), jnp.int32),
            pltpu.VMEM((WIN, CHUNK), jnp.int32),
            pltpu.VMEM((WIN, CHUNK), jnp.int32),
            pltpu.VMEM((CHUNK, D), jnp.float32),
            pltpu.VMEM((CHUNK, D), jnp.float32),
            pltpu.SemaphoreType.DMA,
            pltpu.SemaphoreType.DMA,
            pltpu.SemaphoreType.DMA,
            pltpu.SemaphoreType.DMA,
            pltpu.VMEM_SHARED((N_PAD, D), jnp.float32),
        ],
    )


def _tc1a_body(x_ref, w1_ref, l0w_ref, l0b_ref, h1_ref, ft_ref):
    x = x_ref[...]
    h1_ref[...] = jnp.dot(x, w1_ref[...], preferred_element_type=jnp.float32)
    ft_ref[...] = jnp.maximum(
        jnp.dot(x, l0w_ref[...], preferred_element_type=jnp.float32) + l0b_ref[...],
        0.0,
    )


_tc1a_call = pl.pallas_call(
    _tc1a_body,
    out_shape=(
        jax.ShapeDtypeStruct((N, H), jnp.float32),
        jax.ShapeDtypeStruct((N, H8), jnp.float32),
    ),
)


def _tc1b_body(h1_ref, degp_ref, hp1_ref):
    degt = jnp.transpose(degp_ref[...], (1, 0))
    deg = degt[:N, 0:1] + degt[:N, 1:2] + 1.0
    norm = lax.rsqrt(deg)
    hp1_ref[:N, :] = h1_ref[...] * norm
    hp1_ref[N:, :] = jnp.zeros((N_PAD - N, H), jnp.float32)


_tc1b_call = pl.pallas_call(
    _tc1b_body,
    out_shape=jax.ShapeDtypeStruct((N_PAD, H), jnp.float32),
)


def _tc2_body(aggp_ref, h1_ref, degp_ref, b1_ref, w2_ref, hp2_ref, h2_ref):
    degt = jnp.transpose(degp_ref[...], (1, 0))
    deg = degt[:N, 0:1] + degt[:N, 1:2] + 1.0
    norm = lax.rsqrt(deg)
    inv = 1.0 / deg
    h1 = h1_ref[...]
    agg = (aggp_ref[0, :N, :] + aggp_ref[1, :N, :]) * norm + h1 * inv + b1_ref[...]
    h1f = _leaky(agg)
    h2 = jnp.dot(h1f, w2_ref[...], preferred_element_type=jnp.float32)
    h2_ref[...] = h2
    hp2_ref[:N, :] = h2 * norm
    hp2_ref[N:, :] = jnp.zeros((N_PAD - N, H), jnp.float32)


_tc2_call = pl.pallas_call(
    _tc2_body,
    out_shape=(
        jax.ShapeDtypeStruct((N_PAD, H), jnp.float32),
        jax.ShapeDtypeStruct((N, H), jnp.float32),
    ),
)


def _tc3_body(aggp_ref, h2_ref, degp_ref, b2_ref, batch_ref, ft_ref,
              l1w_ref, l1b_ref, bn1g_ref, bn1b_ref,
              l2w_ref, l2b_ref, bn2g_ref, bn2b_ref, emb_ref, out_ref):
    degt = jnp.transpose(degp_ref[...], (1, 0))
    deg = degt[:N, 0:1] + degt[:N, 1:2] + 1.0
    norm = lax.rsqrt(deg)
    inv = 1.0 / deg
    h2 = h2_ref[...]
    h2f = _leaky((aggp_ref[0, :N, :] + aggp_ref[1, :N, :]) * norm + h2 * inv + b2_ref[...])

    seg = lax.broadcasted_iota(jnp.int32, (G, N), 0)
    pmat = (seg == batch_ref[...]).astype(jnp.float32)
    counts = jnp.sum(pmat, axis=1, keepdims=True)
    denom = jnp.maximum(counts, 1.0)
    pooled = jnp.dot(pmat, h2f, preferred_element_type=jnp.float32) / denom
    ft_pool = jnp.dot(pmat, ft_ref[...], preferred_element_type=jnp.float32) / denom

    z = _leaky(_bn_eval(
        jnp.dot(pooled, l1w_ref[...], preferred_element_type=jnp.float32)
        + l1b_ref[...],
        bn1g_ref[...], bn1b_ref[...]))

    big = counts >= 10.0
    x1 = jnp.where(big, emb_ref[1:2, :], emb_ref[0:1, :])
    x1 = _leaky(_bn_eval(x1, bn2g_ref[...], bn2b_ref[...]))
    x1 = (1.0 / (1.0 + jnp.exp(-x1))) * ft_pool

    cat = jnp.concatenate([z, x1], axis=1)
    out_ref[...] = (
        jnp.dot(cat, l2w_ref[...], preferred_element_type=jnp.float32)
        + l2b_ref[...]
    )


_tc3_call = pl.pallas_call(
    _tc3_body,
    out_shape=jax.ShapeDtypeStruct((G, OUTC), jnp.float32),
)


def kernel(x, edge_index, batch, W_conv1, b_conv1, W_conv2, b_conv2,
           lin0_W, lin0_b, lin1_W, lin1_b, lin2_W, lin2_b,
           bn1_g, bn1_b, bn2_g, bn2_b, emb):
    pad = N + (jnp.arange(E_PAD - E, dtype=jnp.int32) % (N_PAD - N))
    src2 = jnp.concatenate([edge_index[0], pad]).reshape(NW, NCHUNKS * CHUNK)
    dst3 = jnp.concatenate([edge_index[1], pad]).reshape(NW, NCHUNKS, CHUNK)
    ones_c = jnp.ones((CHUNK,), jnp.float32)
    zeros_n = jnp.zeros((N_PAD,), jnp.float32)
    zeros_blk = jnp.zeros((ROWS_PER_SUB, D), jnp.float32)

    degp = _deg_call()(dst3, ones_c, zeros_n)

    h1, ft = _tc1a_call(x, W_conv1, lin0_W, lin0_b.reshape(1, H8))
    hp1 = _tc1b_call(h1, degp)
    agg1 = _agg_call()(hp1, src2, dst3, zeros_blk)
    hp2, h2 = _tc2_call(agg1, h1, degp, b_conv1.reshape(1, H), W_conv2)
    agg2 = _agg_call()(hp2, src2, dst3, zeros_blk)
    return _tc3_call(agg2, h2, degp, b_conv2.reshape(1, H),
                     batch.reshape(1, N), ft,
                     lin1_W, lin1_b.reshape(1, H8),
                     bn1_g.reshape(1, H8), bn1_b.reshape(1, H8),
                     lin2_W, lin2_b.reshape(1, OUTC),
                     bn2_g.reshape(1, H8), bn2_b.reshape(1, H8), emb)
